# baseline (speedup 1.0000x reference)
import jax
import jax.numpy as jnp
from jax import lax
from jax.experimental import pallas as pl
from jax.experimental.pallas import tpu as pltpu
from jax.experimental.pallas import tpu_sc as plsc

_B, _L, _D = 4, 4096, 1024
_NW = 32
_LW = _L // _NW
_CH = 16
_NCH = _LW // _CH
_STEPS = _NCH * _B
_XNB = 5
_PF = 3


def _sc_body(x_hbm, emb_hbm, p_hbm, out_hbm, idx_v, *rest):
    xb = rest[:_XNB]
    eb = rest[_XNB:_XNB + 2]
    xl_sem = rest[_XNB + 2:2 * _XNB + 2]
    st_sem = rest[2 * _XNB + 2:3 * _XNB + 2]
    eg_sem = rest[3 * _XNB + 2:]

    wid = lax.axis_index("s") * 2 + lax.axis_index("c")
    lw0 = wid * _LW

    pltpu.sync_copy(p_hbm.at[pl.ds(wid * _NCH, _NCH)], idx_v)

    def egather(c):
        return pltpu.make_async_copy(emb_hbm.at[idx_v.at[c]], eb[c % 2],
                                     eg_sem[c % 2])

    def xcopy(t, store):
        c, b = t // _B, t % _B
        hbm_slice = out_hbm if store else x_hbm
        hbm_slice = hbm_slice.at[b, pl.ds(lw0 + c * _CH, _CH)]
        buf = xb[t % _XNB]
        sem = (st_sem if store else xl_sem)[t % _XNB]
        if store:
            return pltpu.make_async_copy(buf, hbm_slice, sem)
        return pltpu.make_async_copy(hbm_slice, buf, sem)

    def xstore_half(t, h):
        c, b = t // _B, t % _B
        hbm_slice = out_hbm.at[b, pl.ds(lw0 + c * _CH + h * (_CH // 2),
                                        _CH // 2)]
        buf = xb[t % _XNB].at[pl.ds(h * (_CH // 2), _CH // 2)]
        return pltpu.make_async_copy(buf, hbm_slice, st_sem[t % _XNB])

    egather(0).start()
    egather(1).start()
    for t in range(_PF):
        xcopy(t, False).start()

    for t in range(_STEPS):
        c, b = t // _B, t % _B
        xs, es = t % _XNB, c % 2

        if b == 0:
            egather(c).wait()
        xcopy(t, False).wait()

        xbuf, ebuf = xb[xs], eb[es]

        for h in range(2):
            rbase = h * (_CH // 2)

            @plsc.parallel_loop(0, _CH * _D // 32, 1, unroll=16)
            def _(i, rbase=rbase):
                r = rbase + i // (_D // 16)
                off = (i - (i // (_D // 16)) * (_D // 16)) * 16
                plsc.addupdate(xbuf.at[r, pl.ds(off, 16)],
                               ebuf[r, pl.ds(off, 16)])

            xstore_half(t, h).start()

        if b == _B - 1 and c + 2 < _NCH:
            egather(c + 2).start()
        if t + _PF < _STEPS:
            if t - 2 >= 0:
                for h in range(2):
                    xstore_half(t - 2, h).wait()
            xcopy(t + _PF, False).start()

    for t in range(_STEPS - _XNB, _STEPS):
        for h in range(2):
            xstore_half(t, h).wait()


def kernel(X, embedding, offset):
    B, L, D = X.shape
    P = (jnp.arange(L, dtype=jnp.int32)
         + jnp.asarray(offset, jnp.int32)).reshape(L // _CH, _CH)
    f = pl.kernel(
        _sc_body,
        out_type=jax.ShapeDtypeStruct(X.shape, X.dtype),
        mesh=plsc.VectorSubcoreMesh(core_axis_name="c", subcore_axis_name="s"),
        scratch_types=[
            pltpu.VMEM((_NCH, _CH), jnp.int32),
            *[pltpu.VMEM((_CH, D), jnp.float32) for _ in range(_XNB)],
            *[pltpu.VMEM((_CH, D), jnp.float32) for _ in range(2)],
            *[pltpu.SemaphoreType.DMA for _ in range(2 * _XNB + 2)],
        ],
    )
    return f(X, embedding, P)


# speedup vs baseline: 1.0568x; 1.0568x over previous
import jax
import jax.numpy as jnp
from jax import lax
from jax.experimental import pallas as pl
from jax.experimental.pallas import tpu as pltpu
from jax.experimental.pallas import tpu_sc as plsc

_B, _L, _D = 4, 4096, 1024
_NW = 32
_LW = _L // _NW
_CH = 8
_NCH = _LW // _CH
_STEPS = _NCH * _B
_XNB = 10
_PF = 6
_SG = _XNB - _PF


def _sc_body(x_hbm, emb_hbm, p_hbm, out_hbm, idx_v, *rest):
    xb = rest[:_XNB]
    eb = rest[_XNB:_XNB + 2]
    xl_sem = rest[_XNB + 2:2 * _XNB + 2]
    st_sem = rest[2 * _XNB + 2:3 * _XNB + 2]
    eg_sem = rest[3 * _XNB + 2:]

    wid = lax.axis_index("s") * 2 + lax.axis_index("c")
    lw0 = wid * _LW

    pltpu.sync_copy(p_hbm.at[pl.ds(wid * _NCH, _NCH)], idx_v)

    def egather(c):
        return pltpu.make_async_copy(emb_hbm.at[idx_v.at[c]], eb[c % 2],
                                     eg_sem[c % 2])

    def xcopy(t, store):
        c, b = t // _B, t % _B
        hbm_slice = out_hbm if store else x_hbm
        hbm_slice = hbm_slice.at[b, pl.ds(lw0 + c * _CH, _CH)]
        buf = xb[t % _XNB]
        sem = (st_sem if store else xl_sem)[t % _XNB]
        if store:
            return pltpu.make_async_copy(buf, hbm_slice, sem)
        return pltpu.make_async_copy(hbm_slice, buf, sem)

    egather(0).start()
    egather(1).start()
    for t in range(_PF):
        xcopy(t, False).start()

    for t in range(_STEPS):
        c, b = t // _B, t % _B
        xs, es = t % _XNB, c % 2

        if b == 0:
            egather(c).wait()
        xcopy(t, False).wait()

        xbuf, ebuf = xb[xs], eb[es]

        @plsc.parallel_loop(0, _CH * _D // 16, 1, unroll=8)
        def _(i):
            r = i // (_D // 16)
            off = (i - r * (_D // 16)) * 16
            plsc.addupdate(xbuf.at[r, pl.ds(off, 16)],
                           ebuf[r, pl.ds(off, 16)])

        xcopy(t, True).start()

        if b == _B - 1 and c + 2 < _NCH:
            egather(c + 2).start()
        if t + _PF < _STEPS:
            if t - _SG >= 0:
                xcopy(t - _SG, True).wait()
            xcopy(t + _PF, False).start()

    for t in range(_STEPS - _XNB, _STEPS):
        xcopy(t, True).wait()


def kernel(X, embedding, offset):
    B, L, D = X.shape
    P = (jnp.arange(L, dtype=jnp.int32)
         + jnp.asarray(offset, jnp.int32)).reshape(L // _CH, _CH)
    f = pl.kernel(
        _sc_body,
        out_type=jax.ShapeDtypeStruct(X.shape, X.dtype),
        mesh=plsc.VectorSubcoreMesh(core_axis_name="c", subcore_axis_name="s"),
        scratch_types=[
            pltpu.VMEM((_NCH, _CH), jnp.int32),
            *[pltpu.VMEM((_CH, D), jnp.float32) for _ in range(_XNB)],
            *[pltpu.VMEM((_CH, D), jnp.float32) for _ in range(2)],
            *[pltpu.SemaphoreType.DMA for _ in range(2 * _XNB + 2)],
        ],
    )
    return f(X, embedding, P)


# speedup vs baseline: 1.0812x; 1.0231x over previous
import jax
import jax.numpy as jnp
from jax import lax
from jax.experimental import pallas as pl
from jax.experimental.pallas import tpu as pltpu
from jax.experimental.pallas import tpu_sc as plsc

_B, _L, _D = 4, 4096, 1024
_NW = 32
_LW = _L // _NW
_CH = 16
_NCH = _LW // _CH
_STEPS = _NCH * _B
_XNB = 5
_PF = 3


def _sc_body(x_hbm, emb_hbm, p_hbm, out_hbm, idx_v, *rest):
    xb = rest[:_XNB]
    eb = rest[_XNB:_XNB + 2]
    xl_sem = rest[_XNB + 2:2 * _XNB + 2]
    st_sem = rest[2 * _XNB + 2:3 * _XNB + 2]
    eg_sem = rest[3 * _XNB + 2:]

    wid = lax.axis_index("s") * 2 + lax.axis_index("c")
    lw0 = wid * _LW

    pltpu.sync_copy(p_hbm.at[pl.ds(wid * _NCH, _NCH)], idx_v)

    def egather(c):
        return pltpu.make_async_copy(emb_hbm.at[idx_v.at[c]], eb[c % 2],
                                     eg_sem[c % 2])

    def xcopy(t, store):
        c, b = t // _B, t % _B
        hbm_slice = out_hbm if store else x_hbm
        hbm_slice = hbm_slice.at[b, pl.ds(lw0 + c * _CH, _CH)]
        buf = xb[t % _XNB]
        sem = (st_sem if store else xl_sem)[t % _XNB]
        if store:
            return pltpu.make_async_copy(buf, hbm_slice, sem)
        return pltpu.make_async_copy(hbm_slice, buf, sem)

    egather(0).start()
    egather(1).start()
    for t in range(_PF):
        xcopy(t, False).start()

    for t in range(_STEPS):
        c, b = t // _B, t % _B
        xs, es = t % _XNB, c % 2

        if b == 0:
            egather(c).wait()
        xcopy(t, False).wait()

        xbuf, ebuf = xb[xs], eb[es]

        @plsc.parallel_loop(0, _CH * _D // 16, 1, unroll=8)
        def _(i):
            r = i // (_D // 16)
            off = (i - r * (_D // 16)) * 16
            plsc.addupdate(xbuf.at[r, pl.ds(off, 16)],
                           ebuf[r, pl.ds(off, 16)])

        xcopy(t, True).start()

        if b == _B - 1 and c + 2 < _NCH:
            egather(c + 2).start()
        if t + _PF < _STEPS:
            if t - 2 >= 0:
                xcopy(t - 2, True).wait()
            xcopy(t + _PF, False).start()

    for t in range(_STEPS - _XNB, _STEPS):
        xcopy(t, True).wait()


def kernel(X, embedding, offset):
    B, L, D = X.shape
    P = (jnp.arange(L, dtype=jnp.int32)
         + jnp.asarray(offset, jnp.int32)).reshape(L // _CH, _CH)
    f = pl.kernel(
        _sc_body,
        out_type=jax.ShapeDtypeStruct(X.shape, X.dtype),
        mesh=plsc.VectorSubcoreMesh(core_axis_name="c", subcore_axis_name="s"),
        scratch_types=[
            pltpu.VMEM((_NCH, _CH), jnp.int32),
            *[pltpu.VMEM((_CH, D), jnp.float32) for _ in range(_XNB)],
            *[pltpu.VMEM((_CH, D), jnp.float32) for _ in range(2)],
            *[pltpu.SemaphoreType.DMA for _ in range(2 * _XNB + 2)],
        ],
    )
    return f(X, embedding, P)
